# Initial kernel scaffold; baseline (speedup 1.0000x reference)
import jax
import jax.numpy as jnp
from jax.experimental import pallas as pl


def kernel(upos_encoded, embedding_weight):
    raise NotImplementedError("write your pallas kernel here")


# speedup vs baseline: 3.1936x; 3.1936x over previous
import functools

import jax
import jax.numpy as jnp
from jax import lax
from jax.experimental import pallas as pl
from jax.experimental.pallas import tpu as pltpu
from jax.experimental.pallas import tpu_sc as plsc

VOCAB = 1000
D = 64
N = 4096 * 200

_info = plsc.get_sparse_core_info()
NC, NS = _info.num_cores, _info.num_subcores
NW = NC * NS
NPW = N // NW

K = 128
STEPS = NPW // K


def _emb_kernel(idx_hbm, table_hbm, out_hbm, idx_v, rows_v, sem):
    wid = lax.axis_index("s") * NC + lax.axis_index("c")
    base = wid * NPW

    def step(g, carry):
        off = base + g * K
        pltpu.sync_copy(idx_hbm.at[pl.ds(off, K)], idx_v)
        pltpu.async_copy(table_hbm.at[idx_v], rows_v, sem).wait()
        pltpu.sync_copy(rows_v, out_hbm.at[pl.ds(off, K)])
        return carry

    lax.fori_loop(0, STEPS, step, 0)


@jax.jit
def _emb(idx_flat, table):
    mesh = plsc.VectorSubcoreMesh(core_axis_name="c", subcore_axis_name="s")
    run = functools.partial(
        pl.kernel,
        out_type=jax.ShapeDtypeStruct((N, D), jnp.float32),
        mesh=mesh,
        scratch_types=[
            pltpu.VMEM((K,), jnp.int32),
            pltpu.VMEM((K, D), jnp.float32),
            pltpu.SemaphoreType.DMA,
        ],
        compiler_params=pltpu.CompilerParams(use_tc_tiling_on_sc=False),
    )(_emb_kernel)
    return run(idx_flat, table)


def kernel(upos_encoded, embedding_weight):
    idx_flat = upos_encoded.reshape(N).astype(jnp.int32)
    out = _emb(idx_flat, embedding_weight)
    return out.reshape(upos_encoded.shape + (D,))


# speedup vs baseline: 3.5998x; 1.1272x over previous
import functools

import jax
import jax.numpy as jnp
from jax import lax
from jax.experimental import pallas as pl
from jax.experimental.pallas import tpu as pltpu
from jax.experimental.pallas import tpu_sc as plsc

VOCAB = 1000
D = 64
N = 4096 * 200

_info = plsc.get_sparse_core_info()
NC, NS = _info.num_cores, _info.num_subcores
NW = NC * NS
NPW = N // NW

K = 128
KG = 4
M = K * KG
NSUP = NPW // M
NROWS = NPW // K


def _emb_kernel(idx_hbm, table_hbm, out_hbm, idx_v, x0, x1, gs0, gs1, os0, os1):
    wid = lax.axis_index("s") * NC + lax.axis_index("c")
    base = wid * NPW
    pltpu.sync_copy(idx_hbm.at[pl.ds(wid * NROWS, NROWS)], idx_v)

    bufs = (x0, x1)
    gsems = (gs0, gs1)
    osems = (os0, os1)

    def fire(s, p):
        descs = []
        for j in range(KG):
            descs.append(
                pltpu.async_copy(
                    table_hbm.at[idx_v.at[s * KG + j]],
                    bufs[p].at[pl.ds(j * K, K)],
                    gsems[p],
                )
            )
        return descs

    def start_out(s, p):
        return pltpu.async_copy(bufs[p], out_hbm.at[pl.ds(base + s * M, M)], osems[p])

    def do_chunk(s, p):
        for d in fire(s, p):
            d.wait()
        start_out(s, p)

    do_chunk(0, 0)
    do_chunk(1, 1)

    def outer(t, carry):
        s0 = 2 + 2 * t
        for p in range(2):
            s = s0 + p
            pltpu.make_async_copy(
                bufs[p], out_hbm.at[pl.ds(base + (s - 2) * M, M)], osems[p]
            ).wait()
            do_chunk(s, p)
        return carry

    lax.fori_loop(0, (NSUP - 2) // 2, outer, 0)

    for p in range(2):
        pltpu.make_async_copy(
            bufs[p], out_hbm.at[pl.ds(base + (NSUP - 2 + p) * M, M)], osems[p]
        ).wait()


@jax.jit
def _emb(idx_2d, table):
    mesh = plsc.VectorSubcoreMesh(core_axis_name="c", subcore_axis_name="s")
    run = functools.partial(
        pl.kernel,
        out_type=jax.ShapeDtypeStruct((N, D), jnp.float32),
        mesh=mesh,
        scratch_types=[
            pltpu.VMEM((NROWS, K), jnp.int32),
            pltpu.VMEM((M, D), jnp.float32),
            pltpu.VMEM((M, D), jnp.float32),
            pltpu.SemaphoreType.DMA,
            pltpu.SemaphoreType.DMA,
            pltpu.SemaphoreType.DMA,
            pltpu.SemaphoreType.DMA,
        ],
        compiler_params=pltpu.CompilerParams(use_tc_tiling_on_sc=False),
    )(_emb_kernel)
    return run(idx_2d, table)


def kernel(upos_encoded, embedding_weight):
    idx_2d = upos_encoded.reshape(N // K, K).astype(jnp.int32)
    out = _emb(idx_2d, embedding_weight)
    return out.reshape(upos_encoded.shape + (D,))
